# Initial kernel scaffold; baseline (speedup 1.0000x reference)
import jax
import jax.numpy as jnp
from jax.experimental import pallas as pl


def kernel(x, edge_index, W1, b1, W2, b2):
    raise NotImplementedError("write your pallas kernel here")


# speedup vs baseline: 7.8527x; 7.8527x over previous
import functools

import jax
import jax.numpy as jnp
from jax import lax
from jax.experimental import pallas as pl
from jax.experimental.pallas import tpu as pltpu
from jax.experimental.pallas import tpu_sc as plsc

N_NODES = 10000
NP = 10240
E = 320000
NW = 32
CHUNK = 128
NCHUNK = 80
EP = NW * NCHUNK * CHUNK
RPT = NP // 16
BLK = 512
NBLK = NP // BLK


def _seg_sum_sc(width):
  mesh = plsc.VectorSubcoreMesh(core_axis_name="c", subcore_axis_name="s")

  @functools.partial(
      pl.kernel,
      out_type=jax.ShapeDtypeStruct((2, NP, width), jnp.float32),
      mesh=mesh,
      compiler_params=pltpu.CompilerParams(use_tc_tiling_on_sc=False),
      scratch_types=[
          pltpu.VMEM((NCHUNK, CHUNK), jnp.int32),
          pltpu.VMEM((NCHUNK, CHUNK), jnp.int32),
          pltpu.VMEM((2, CHUNK, width), jnp.float32),
          pltpu.VMEM_SHARED((NP, width), jnp.float32),
          pltpu.SemaphoreType.DMA((2,)),
      ],
  )
  def seg(table_hbm, src_hbm, dst_hbm, zeros_hbm, out_hbm,
          src_v, dst_v, rows_v, acc_sh, sem):
    cid = lax.axis_index("c")
    sid = lax.axis_index("s")
    wid = sid * 2 + cid
    pltpu.sync_copy(zeros_hbm.at[pl.ds(sid * RPT, RPT)],
                    acc_sh.at[pl.ds(sid * RPT, RPT)])
    pltpu.sync_copy(src_hbm.at[wid], src_v)
    pltpu.sync_copy(dst_hbm.at[wid], dst_v)
    plsc.subcore_barrier()

    pltpu.async_copy(table_hbm.at[src_v.at[0]], rows_v.at[0], sem.at[0])

    def body(j, carry):
      @pl.when(j + 1 < NCHUNK)
      def _():
        pltpu.async_copy(table_hbm.at[src_v.at[j + 1]],
                         rows_v.at[(j + 1) % 2], sem.at[(j + 1) % 2])
      pltpu.make_async_copy(table_hbm.at[src_v.at[j]], rows_v.at[j % 2],
                            sem.at[j % 2]).wait()
      pltpu.sync_copy(rows_v.at[j % 2], acc_sh.at[dst_v.at[j]], add=True)
      return carry

    lax.fori_loop(0, NCHUNK, body, 0)
    plsc.subcore_barrier()
    pltpu.sync_copy(acc_sh.at[pl.ds(sid * RPT, RPT)],
                    out_hbm.at[cid, pl.ds(sid * RPT, RPT)])

  return seg


def _tc1(x_pad, W1a, W1b):
  def body(x_ref, wa_ref, wb_ref, xa_ref, xb_ref):
    x = x_ref[...]
    xa_ref[...] = jnp.dot(x, wa_ref[...], preferred_element_type=jnp.float32)
    xb_ref[...] = jnp.dot(x, wb_ref[...], preferred_element_type=jnp.float32)

  return pl.pallas_call(
      body,
      grid=(NBLK,),
      in_specs=[
          pl.BlockSpec((BLK, 128), lambda i: (i, 0)),
          pl.BlockSpec((128, 64), lambda i: (0, 0)),
          pl.BlockSpec((128, 64), lambda i: (0, 0)),
      ],
      out_specs=[
          pl.BlockSpec((BLK, 64), lambda i: (i, 0)),
          pl.BlockSpec((BLK, 64), lambda i: (i, 0)),
      ],
      out_shape=[jax.ShapeDtypeStruct((NP, 64), jnp.float32)] * 2,
  )(x_pad, W1a, W1b)


def _tc2(xa, p, b1r, W2a, W2b, b2r):
  def body(xa_ref, p_ref, b1_ref, wa_ref, wb_ref, b2_ref, ya_ref, hb_ref):
    h = xa_ref[...] + p_ref[0] + p_ref[1] + b1_ref[...]
    h = jnp.maximum(h, 0.0)
    ya_ref[...] = jnp.dot(h, wa_ref[...],
                          preferred_element_type=jnp.float32) + b2_ref[...]
    hb_ref[...] = jnp.dot(h, wb_ref[...], preferred_element_type=jnp.float32)

  return pl.pallas_call(
      body,
      grid=(NBLK,),
      in_specs=[
          pl.BlockSpec((BLK, 64), lambda i: (i, 0)),
          pl.BlockSpec((2, BLK, 64), lambda i: (0, i, 0)),
          pl.BlockSpec((1, 64), lambda i: (0, 0)),
          pl.BlockSpec((64, 16), lambda i: (0, 0)),
          pl.BlockSpec((64, 16), lambda i: (0, 0)),
          pl.BlockSpec((1, 16), lambda i: (0, 0)),
      ],
      out_specs=[
          pl.BlockSpec((BLK, 16), lambda i: (i, 0)),
          pl.BlockSpec((BLK, 16), lambda i: (i, 0)),
      ],
      out_shape=[jax.ShapeDtypeStruct((NP, 16), jnp.float32)] * 2,
  )(xa, p, b1r, W2a, W2b, b2r)


def _tc3(ya, q):
  def body(ya_ref, q_ref, o_ref):
    o_ref[...] = ya_ref[...] + q_ref[0] + q_ref[1]

  return pl.pallas_call(
      body,
      grid=(NBLK,),
      in_specs=[
          pl.BlockSpec((BLK, 16), lambda i: (i, 0)),
          pl.BlockSpec((2, BLK, 16), lambda i: (0, i, 0)),
      ],
      out_specs=pl.BlockSpec((BLK, 16), lambda i: (i, 0)),
      out_shape=jax.ShapeDtypeStruct((NP, 16), jnp.float32),
  )(ya, q)


def kernel(x, edge_index, W1, b1, W2, b2):
  src = edge_index[0].astype(jnp.int32)
  dst = edge_index[1].astype(jnp.int32)
  pad_e = EP - E
  src3 = jnp.concatenate([src, jnp.zeros((pad_e,), jnp.int32)]).reshape(
      NW, NCHUNK, CHUNK)
  dst3 = jnp.concatenate([dst, jnp.full((pad_e,), N_NODES, jnp.int32)]
                         ).reshape(NW, NCHUNK, CHUNK)
  x_pad = jnp.pad(x, ((0, NP - N_NODES), (0, 0)))
  W1a, W1b = W1[:128], W1[128:]
  W2a = jnp.pad(W2[:64], ((0, 0), (0, 11)))
  W2b = jnp.pad(W2[64:], ((0, 0), (0, 11)))
  b1r = b1.reshape(1, 64)
  b2r = jnp.pad(b2, (0, 11)).reshape(1, 16)
  zeros64 = jnp.zeros((NP, 64), jnp.float32)
  zeros16 = jnp.zeros((NP, 16), jnp.float32)

  xa, xb = _tc1(x_pad, W1a, W1b)
  p = _seg_sum_sc(64)(xb, src3, dst3, zeros64)
  ya, hb = _tc2(xa, p, b1r, W2a, W2b, b2r)
  q = _seg_sum_sc(16)(hb, src3, dst3, zeros16)
  o = _tc3(ya, q)
  return o[:N_NODES, :5]


# speedup vs baseline: 14.7669x; 1.8805x over previous
import functools

import jax
import jax.numpy as jnp
from jax import lax
from jax.experimental import pallas as pl
from jax.experimental.pallas import tpu as pltpu
from jax.experimental.pallas import tpu_sc as plsc

N_NODES = 10000
NP = 10240
E = 320000
NW = 32
CHUNK = 128
NCHUNK = 80
EP = NW * NCHUNK * CHUNK
RPT = NP // 16
BLK = 512
NBLK = NP // BLK


def _seg_sum_sc(width):
  mesh = plsc.VectorSubcoreMesh(core_axis_name="c", subcore_axis_name="s")

  @functools.partial(
      pl.kernel,
      out_type=jax.ShapeDtypeStruct((2, NP, width), jnp.float32),
      mesh=mesh,
      compiler_params=pltpu.CompilerParams(use_tc_tiling_on_sc=False),
      scratch_types=[
          pltpu.VMEM((NCHUNK, CHUNK), jnp.int32),
          pltpu.VMEM((NCHUNK, CHUNK), jnp.int32),
          pltpu.VMEM((2, CHUNK, width), jnp.float32),
          pltpu.VMEM_SHARED((NP, width), jnp.float32),
          pltpu.VMEM_SHARED((NP, width), jnp.float32),
          pltpu.SemaphoreType.DMA((2,)),
      ],
  )
  def seg(table_hbm, src_hbm, dst_hbm, zeros_hbm, out_hbm,
          src_v, dst_v, rows_v, acc_sh, table_sh, sem):
    cid = lax.axis_index("c")
    sid = lax.axis_index("s")
    wid = sid * 2 + cid
    pltpu.sync_copy(zeros_hbm.at[pl.ds(sid * RPT, RPT)],
                    acc_sh.at[pl.ds(sid * RPT, RPT)])
    pltpu.sync_copy(table_hbm.at[pl.ds(sid * RPT, RPT)],
                    table_sh.at[pl.ds(sid * RPT, RPT)])
    pltpu.sync_copy(src_hbm.at[wid], src_v)
    pltpu.sync_copy(dst_hbm.at[wid], dst_v)
    plsc.subcore_barrier()

    pltpu.async_copy(table_sh.at[src_v.at[0]], rows_v.at[0], sem.at[0])

    def body(j, carry):
      @pl.when(j + 1 < NCHUNK)
      def _():
        pltpu.async_copy(table_sh.at[src_v.at[j + 1]],
                         rows_v.at[(j + 1) % 2], sem.at[(j + 1) % 2])
      pltpu.make_async_copy(table_sh.at[src_v.at[j]], rows_v.at[j % 2],
                            sem.at[j % 2]).wait()
      pltpu.sync_copy(rows_v.at[j % 2], acc_sh.at[dst_v.at[j]], add=True)
      return carry

    lax.fori_loop(0, NCHUNK, body, 0)
    plsc.subcore_barrier()
    pltpu.sync_copy(acc_sh.at[pl.ds(sid * RPT, RPT)],
                    out_hbm.at[cid, pl.ds(sid * RPT, RPT)])

  return seg


def _tc1(x_pad, W1a, W1b):
  def body(x_ref, wa_ref, wb_ref, xa_ref, xb_ref):
    x = x_ref[...]
    xa_ref[...] = jnp.dot(x, wa_ref[...], preferred_element_type=jnp.float32)
    xb_ref[...] = jnp.dot(x, wb_ref[...], preferred_element_type=jnp.float32)

  return pl.pallas_call(
      body,
      grid=(NBLK,),
      in_specs=[
          pl.BlockSpec((BLK, 128), lambda i: (i, 0)),
          pl.BlockSpec((128, 64), lambda i: (0, 0)),
          pl.BlockSpec((128, 64), lambda i: (0, 0)),
      ],
      out_specs=[
          pl.BlockSpec((BLK, 64), lambda i: (i, 0)),
          pl.BlockSpec((BLK, 64), lambda i: (i, 0)),
      ],
      out_shape=[jax.ShapeDtypeStruct((NP, 64), jnp.float32)] * 2,
  )(x_pad, W1a, W1b)


def _tc2(xa, p, b1r, W2a, W2b, b2r):
  def body(xa_ref, p_ref, b1_ref, wa_ref, wb_ref, b2_ref, ya_ref, hb_ref):
    h = xa_ref[...] + p_ref[0] + p_ref[1] + b1_ref[...]
    h = jnp.maximum(h, 0.0)
    ya_ref[...] = jnp.dot(h, wa_ref[...],
                          preferred_element_type=jnp.float32) + b2_ref[...]
    hb_ref[...] = jnp.dot(h, wb_ref[...], preferred_element_type=jnp.float32)

  return pl.pallas_call(
      body,
      grid=(NBLK,),
      in_specs=[
          pl.BlockSpec((BLK, 64), lambda i: (i, 0)),
          pl.BlockSpec((2, BLK, 64), lambda i: (0, i, 0)),
          pl.BlockSpec((1, 64), lambda i: (0, 0)),
          pl.BlockSpec((64, 16), lambda i: (0, 0)),
          pl.BlockSpec((64, 16), lambda i: (0, 0)),
          pl.BlockSpec((1, 16), lambda i: (0, 0)),
      ],
      out_specs=[
          pl.BlockSpec((BLK, 16), lambda i: (i, 0)),
          pl.BlockSpec((BLK, 16), lambda i: (i, 0)),
      ],
      out_shape=[jax.ShapeDtypeStruct((NP, 16), jnp.float32)] * 2,
  )(xa, p, b1r, W2a, W2b, b2r)


def _tc3(ya, q):
  def body(ya_ref, q_ref, o_ref):
    o_ref[...] = ya_ref[...] + q_ref[0] + q_ref[1]

  return pl.pallas_call(
      body,
      grid=(NBLK,),
      in_specs=[
          pl.BlockSpec((BLK, 16), lambda i: (i, 0)),
          pl.BlockSpec((2, BLK, 16), lambda i: (0, i, 0)),
      ],
      out_specs=pl.BlockSpec((BLK, 16), lambda i: (i, 0)),
      out_shape=jax.ShapeDtypeStruct((NP, 16), jnp.float32),
  )(ya, q)


def kernel(x, edge_index, W1, b1, W2, b2):
  src = edge_index[0].astype(jnp.int32)
  dst = edge_index[1].astype(jnp.int32)
  pad_e = EP - E
  src3 = jnp.concatenate([src, jnp.zeros((pad_e,), jnp.int32)]).reshape(
      NW, NCHUNK, CHUNK)
  dst3 = jnp.concatenate([dst, jnp.full((pad_e,), N_NODES, jnp.int32)]
                         ).reshape(NW, NCHUNK, CHUNK)
  x_pad = jnp.pad(x, ((0, NP - N_NODES), (0, 0)))
  W1a, W1b = W1[:128], W1[128:]
  W2a = jnp.pad(W2[:64], ((0, 0), (0, 11)))
  W2b = jnp.pad(W2[64:], ((0, 0), (0, 11)))
  b1r = b1.reshape(1, 64)
  b2r = jnp.pad(b2, (0, 11)).reshape(1, 16)
  zeros64 = jnp.zeros((NP, 64), jnp.float32)
  zeros16 = jnp.zeros((NP, 16), jnp.float32)

  xa, xb = _tc1(x_pad, W1a, W1b)
  p = _seg_sum_sc(64)(xb, src3, dst3, zeros64)
  ya, hb = _tc2(xa, p, b1r, W2a, W2b, b2r)
  q = _seg_sum_sc(16)(hb, src3, dst3, zeros16)
  o = _tc3(ya, q)
  return o[:N_NODES, :5]


# speedup vs baseline: 15.4754x; 1.0480x over previous
import functools

import jax
import jax.numpy as jnp
from jax import lax
from jax.experimental import pallas as pl
from jax.experimental.pallas import tpu as pltpu
from jax.experimental.pallas import tpu_sc as plsc

N_NODES = 10000
NP = 10240
E = 320000
CHUNK = 128
EP = 327680
RPT = NP // 16
BLK = 512
NBLK = NP // BLK


def _seg_sum_colsplit(table2, src3, dst3, zeros):
  mesh = plsc.VectorSubcoreMesh(core_axis_name="c", subcore_axis_name="s")
  nch = 160

  @functools.partial(
      pl.kernel,
      out_type=jax.ShapeDtypeStruct((2, NP, 32), jnp.float32),
      mesh=mesh,
      compiler_params=pltpu.CompilerParams(use_tc_tiling_on_sc=False),
      scratch_types=[
          pltpu.VMEM((nch, CHUNK), jnp.int32),
          pltpu.VMEM((nch, CHUNK), jnp.int32),
          pltpu.VMEM((2, CHUNK, 32), jnp.float32),
          pltpu.VMEM_SHARED((NP, 32), jnp.float32),
          pltpu.VMEM_SHARED((NP, 32), jnp.float32),
          pltpu.SemaphoreType.DMA((2,)),
      ],
  )
  def seg(table_hbm, src_hbm, dst_hbm, zeros_hbm, out_hbm,
          src_v, dst_v, rows_v, acc_sh, table_sh, sem):
    cid = lax.axis_index("c")
    sid = lax.axis_index("s")
    pltpu.sync_copy(zeros_hbm.at[pl.ds(sid * RPT, RPT)],
                    acc_sh.at[pl.ds(sid * RPT, RPT)])
    pltpu.sync_copy(table_hbm.at[cid, pl.ds(sid * RPT, RPT)],
                    table_sh.at[pl.ds(sid * RPT, RPT)])
    pltpu.sync_copy(src_hbm.at[sid], src_v)
    pltpu.sync_copy(dst_hbm.at[sid], dst_v)
    plsc.subcore_barrier()

    pltpu.async_copy(table_sh.at[src_v.at[0]], rows_v.at[0], sem.at[0])

    def body(j, carry):
      @pl.when(j + 1 < nch)
      def _():
        pltpu.async_copy(table_sh.at[src_v.at[j + 1]],
                         rows_v.at[(j + 1) % 2], sem.at[(j + 1) % 2])
      pltpu.make_async_copy(table_sh.at[src_v.at[j]], rows_v.at[j % 2],
                            sem.at[j % 2]).wait()
      pltpu.sync_copy(rows_v.at[j % 2], acc_sh.at[dst_v.at[j]], add=True)
      return carry

    lax.fori_loop(0, nch, body, 0)
    plsc.subcore_barrier()
    pltpu.sync_copy(acc_sh.at[pl.ds(sid * RPT, RPT)],
                    out_hbm.at[cid, pl.ds(sid * RPT, RPT)])

  return seg(table2, src3, dst3, zeros)


def _seg_sum_edgesplit(table, src3, dst3, zeros):
  mesh = plsc.VectorSubcoreMesh(core_axis_name="c", subcore_axis_name="s")
  nch = 80

  @functools.partial(
      pl.kernel,
      out_type=jax.ShapeDtypeStruct((2, NP, 8), jnp.float32),
      mesh=mesh,
      compiler_params=pltpu.CompilerParams(use_tc_tiling_on_sc=False),
      scratch_types=[
          pltpu.VMEM((nch, CHUNK), jnp.int32),
          pltpu.VMEM((nch, CHUNK), jnp.int32),
          pltpu.VMEM((2, CHUNK, 8), jnp.float32),
          pltpu.VMEM_SHARED((NP, 8), jnp.float32),
          pltpu.VMEM_SHARED((NP, 8), jnp.float32),
          pltpu.SemaphoreType.DMA((2,)),
      ],
  )
  def seg(table_hbm, src_hbm, dst_hbm, zeros_hbm, out_hbm,
          src_v, dst_v, rows_v, acc_sh, table_sh, sem):
    cid = lax.axis_index("c")
    sid = lax.axis_index("s")
    wid = sid * 2 + cid
    pltpu.sync_copy(zeros_hbm.at[pl.ds(sid * RPT, RPT)],
                    acc_sh.at[pl.ds(sid * RPT, RPT)])
    pltpu.sync_copy(table_hbm.at[pl.ds(sid * RPT, RPT)],
                    table_sh.at[pl.ds(sid * RPT, RPT)])
    pltpu.sync_copy(src_hbm.at[wid], src_v)
    pltpu.sync_copy(dst_hbm.at[wid], dst_v)
    plsc.subcore_barrier()

    pltpu.async_copy(table_sh.at[src_v.at[0]], rows_v.at[0], sem.at[0])

    def body(j, carry):
      @pl.when(j + 1 < nch)
      def _():
        pltpu.async_copy(table_sh.at[src_v.at[j + 1]],
                         rows_v.at[(j + 1) % 2], sem.at[(j + 1) % 2])
      pltpu.make_async_copy(table_sh.at[src_v.at[j]], rows_v.at[j % 2],
                            sem.at[j % 2]).wait()
      pltpu.sync_copy(rows_v.at[j % 2], acc_sh.at[dst_v.at[j]], add=True)
      return carry

    lax.fori_loop(0, nch, body, 0)
    plsc.subcore_barrier()
    pltpu.sync_copy(acc_sh.at[pl.ds(sid * RPT, RPT)],
                    out_hbm.at[cid, pl.ds(sid * RPT, RPT)])

  return seg(table, src3, dst3, zeros)


def _tc1(x, W1a, W1b2):
  def body(x_ref, wa_ref, wb_ref, xa_ref, xb_ref):
    xv = x_ref[...]
    xa_ref[...] = jnp.dot(xv, wa_ref[...], preferred_element_type=jnp.float32)
    xb_ref[0] = jnp.dot(xv, wb_ref[0], preferred_element_type=jnp.float32)
    xb_ref[1] = jnp.dot(xv, wb_ref[1], preferred_element_type=jnp.float32)

  return pl.pallas_call(
      body,
      grid=(NBLK,),
      in_specs=[
          pl.BlockSpec((BLK, 128), lambda i: (i, 0)),
          pl.BlockSpec((128, 64), lambda i: (0, 0)),
          pl.BlockSpec((2, 128, 32), lambda i: (0, 0, 0)),
      ],
      out_specs=[
          pl.BlockSpec((BLK, 64), lambda i: (i, 0)),
          pl.BlockSpec((2, BLK, 32), lambda i: (0, i, 0)),
      ],
      out_shape=[
          jax.ShapeDtypeStruct((NP, 64), jnp.float32),
          jax.ShapeDtypeStruct((2, NP, 32), jnp.float32),
      ],
  )(x, W1a, W1b2)


def _tc2(xa, p, b1r, W2a, W2b, b2r):
  def body(xa_ref, p_ref, b1_ref, wa_ref, wb_ref, b2_ref, ya_ref, hb_ref):
    h = xa_ref[...] + jnp.concatenate([p_ref[0], p_ref[1]], axis=1)
    h = jnp.maximum(h + b1_ref[...], 0.0)
    ya_ref[...] = jnp.dot(h, wa_ref[...],
                          preferred_element_type=jnp.float32) + b2_ref[...]
    hb_ref[...] = jnp.dot(h, wb_ref[...], preferred_element_type=jnp.float32)

  return pl.pallas_call(
      body,
      grid=(NBLK,),
      in_specs=[
          pl.BlockSpec((BLK, 64), lambda i: (i, 0)),
          pl.BlockSpec((2, BLK, 32), lambda i: (0, i, 0)),
          pl.BlockSpec((1, 64), lambda i: (0, 0)),
          pl.BlockSpec((64, 8), lambda i: (0, 0)),
          pl.BlockSpec((64, 8), lambda i: (0, 0)),
          pl.BlockSpec((1, 8), lambda i: (0, 0)),
      ],
      out_specs=[
          pl.BlockSpec((BLK, 8), lambda i: (i, 0)),
          pl.BlockSpec((BLK, 8), lambda i: (i, 0)),
      ],
      out_shape=[jax.ShapeDtypeStruct((NP, 8), jnp.float32)] * 2,
  )(xa, p, b1r, W2a, W2b, b2r)


def _tc3(ya, q):
  def body(ya_ref, q_ref, o_ref):
    o_ref[...] = (ya_ref[...] + q_ref[0] + q_ref[1])[:, :5]

  return pl.pallas_call(
      body,
      grid=(NBLK,),
      in_specs=[
          pl.BlockSpec((BLK, 8), lambda i: (i, 0)),
          pl.BlockSpec((2, BLK, 8), lambda i: (0, i, 0)),
      ],
      out_specs=pl.BlockSpec((BLK, 5), lambda i: (i, 0)),
      out_shape=jax.ShapeDtypeStruct((N_NODES, 5), jnp.float32),
  )(ya, q)


def kernel(x, edge_index, W1, b1, W2, b2):
  src = edge_index[0].astype(jnp.int32)
  dst = edge_index[1].astype(jnp.int32)
  pad_e = EP - E
  src_f = jnp.concatenate([src, jnp.zeros((pad_e,), jnp.int32)])
  dst_f = jnp.concatenate([dst, jnp.full((pad_e,), N_NODES, jnp.int32)])
  src16 = src_f.reshape(16, 160, CHUNK)
  dst16 = dst_f.reshape(16, 160, CHUNK)
  src32 = src_f.reshape(32, 80, CHUNK)
  dst32 = dst_f.reshape(32, 80, CHUNK)
  W1a = W1[:128]
  W1b2 = jnp.stack([W1[128:, :32], W1[128:, 32:]])
  W2a = jnp.pad(W2[:64], ((0, 0), (0, 3)))
  W2b = jnp.pad(W2[64:], ((0, 0), (0, 3)))
  b1r = b1.reshape(1, 64)
  b2r = jnp.pad(b2, (0, 3)).reshape(1, 8)
  zeros32 = jnp.zeros((NP, 32), jnp.float32)
  zeros8 = jnp.zeros((NP, 8), jnp.float32)

  xa, xb2 = _tc1(x, W1a, W1b2)
  p = _seg_sum_colsplit(xb2, src16, dst16, zeros32)
  ya, hb = _tc2(xa, p, b1r, W2a, W2b, b2r)
  q = _seg_sum_edgesplit(hb, src32, dst32, zeros8)
  return _tc3(ya, q)
